# Initial kernel scaffold; baseline (speedup 1.0000x reference)
import jax
import jax.numpy as jnp
from jax.experimental import pallas as pl


def kernel(v_feat, t_feat, Wv, bv, Wt, bt, modal_weight, user_emb, item_emb, edge_index, image_original_adj, text_original_adj, build_item_graph):
    raise NotImplementedError("write your pallas kernel here")


# speedup vs baseline: 1.2481x; 1.2481x over previous
import functools

import jax
import jax.numpy as jnp
from jax.experimental import pallas as pl
from jax.experimental.pallas import tpu as pltpu

NUM_USER = 16384
NUM_ITEM = 4096
DIM_E = 64
TOPK = 10
N_LAYERS = 2
LAMBDA_COEFF = 0.9

ROWS_B1 = 256
ROWS_B2 = 256


def _proj_norm_body(v_ref, wv_ref, bv_ref, t_ref, wt_ref, bt_ref, xi_ref, xt_ref):
    fi = jnp.dot(v_ref[:], wv_ref[:], preferred_element_type=jnp.float32) + bv_ref[:]
    ft = jnp.dot(t_ref[:], wt_ref[:], preferred_element_type=jnp.float32) + bt_ref[:]
    xi_ref[:] = fi / jnp.sqrt(jnp.sum(fi * fi, axis=1, keepdims=True))
    xt_ref[:] = ft / jnp.sqrt(jnp.sum(ft * ft, axis=1, keepdims=True))


def _projected_normalized(v_feat, t_feat, Wv, bv, Wt, bt):
    n = v_feat.shape[0]
    rb = 512
    grid = (n // rb,)
    return pl.pallas_call(
        _proj_norm_body,
        grid=grid,
        in_specs=[
            pl.BlockSpec((rb, v_feat.shape[1]), lambda i: (i, 0)),
            pl.BlockSpec((v_feat.shape[1], DIM_E), lambda i: (0, 0)),
            pl.BlockSpec((1, DIM_E), lambda i: (0, 0)),
            pl.BlockSpec((rb, t_feat.shape[1]), lambda i: (i, 0)),
            pl.BlockSpec((t_feat.shape[1], DIM_E), lambda i: (0, 0)),
            pl.BlockSpec((1, DIM_E), lambda i: (0, 0)),
        ],
        out_specs=[
            pl.BlockSpec((rb, DIM_E), lambda i: (i, 0)),
            pl.BlockSpec((rb, DIM_E), lambda i: (i, 0)),
        ],
        out_shape=[
            jax.ShapeDtypeStruct((n, DIM_E), jnp.float32),
            jax.ShapeDtypeStruct((n, DIM_E), jnp.float32),
        ],
    )(v_feat, Wv, bv.reshape(1, DIM_E), t_feat, Wt, bt.reshape(1, DIM_E))


def _row_topk_threshold(sim):
    cur = sim
    for _ in range(TOPK - 1):
        m = jnp.max(cur, axis=1, keepdims=True)
        cur = jnp.where(cur >= m, -jnp.inf, cur)
    return jnp.max(cur, axis=1, keepdims=True)


def _knn_w_body(wvec_ref, xi_blk, xi_all, xt_blk, xt_all, w_out, rowsum_out):
    si = jax.lax.dot_general(xi_blk[:], xi_all[:], (((1,), (1,)), ((), ())),
                             preferred_element_type=jnp.float32)
    st = jax.lax.dot_general(xt_blk[:], xt_all[:], (((1,), (1,)), ((), ())),
                             preferred_element_type=jnp.float32)
    thr_i = _row_topk_threshold(si)
    thr_t = _row_topk_threshold(st)
    w0 = wvec_ref[0, 0]
    w1 = wvec_ref[0, 1]
    w = w0 * jnp.where(si >= thr_i, si, 0.0) + w1 * jnp.where(st >= thr_t, st, 0.0)
    w_out[:] = w
    rowsum_out[0, 0, :] = jnp.sum(w, axis=1)


def _knn_w(xi, xt, wvec):
    n = xi.shape[0]
    grid = (n // ROWS_B1,)
    return pl.pallas_call(
        _knn_w_body,
        grid=grid,
        in_specs=[
            pl.BlockSpec(memory_space=pltpu.SMEM),
            pl.BlockSpec((ROWS_B1, DIM_E), lambda i: (i, 0)),
            pl.BlockSpec((n, DIM_E), lambda i: (0, 0)),
            pl.BlockSpec((ROWS_B1, DIM_E), lambda i: (i, 0)),
            pl.BlockSpec((n, DIM_E), lambda i: (0, 0)),
        ],
        out_specs=[
            pl.BlockSpec((ROWS_B1, n), lambda i: (i, 0)),
            pl.BlockSpec((1, 1, ROWS_B1), lambda i: (i, 0, 0)),
        ],
        out_shape=[
            jax.ShapeDtypeStruct((n, n), jnp.float32),
            jax.ShapeDtypeStruct((n // ROWS_B1, 1, ROWS_B1), jnp.float32),
        ],
    )(wvec.reshape(1, 2), xi, xi, xt, xt)


def _item_h_body(ab_ref, dl_blk, w_blk, io_blk, to_blk, es_ref, e_ref, h_out):
    a = ab_ref[0, 0]
    b = ab_ref[0, 1]
    learned = jnp.dot(w_blk[:], es_ref[:], preferred_element_type=jnp.float32)
    learned = learned * dl_blk[0, 0, :][:, None]
    orig = a * io_blk[:] + b * to_blk[:]
    h_out[:] = learned + jnp.dot(orig, e_ref[:], preferred_element_type=jnp.float32)


def _item_h(w, dl, image_original_adj, text_original_adj, es, item_emb, ab):
    n = w.shape[0]
    grid = (n // ROWS_B2,)
    return pl.pallas_call(
        _item_h_body,
        grid=grid,
        in_specs=[
            pl.BlockSpec(memory_space=pltpu.SMEM),
            pl.BlockSpec((1, 1, ROWS_B2), lambda i: (i, 0, 0)),
            pl.BlockSpec((ROWS_B2, n), lambda i: (i, 0)),
            pl.BlockSpec((ROWS_B2, n), lambda i: (i, 0)),
            pl.BlockSpec((ROWS_B2, n), lambda i: (i, 0)),
            pl.BlockSpec((n, DIM_E), lambda i: (0, 0)),
            pl.BlockSpec((n, DIM_E), lambda i: (0, 0)),
        ],
        out_specs=pl.BlockSpec((ROWS_B2, DIM_E), lambda i: (i, 0)),
        out_shape=jax.ShapeDtypeStruct((n, DIM_E), jnp.float32),
    )(ab.reshape(1, 2), dl.reshape(n // ROWS_B2, 1, ROWS_B2), w,
      image_original_adj, text_original_adj, es, item_emb)


def kernel(v_feat, t_feat, Wv, bv, Wt, bt, modal_weight, user_emb, item_emb,
           edge_index, image_original_adj, text_original_adj, build_item_graph):
    weight = jax.nn.softmax(modal_weight, axis=0)

    xi, xt = _projected_normalized(v_feat, t_feat, Wv, bv, Wt, bt)
    w, rowsum3 = _knn_w(xi, xt, weight)
    rowsum = rowsum3.reshape(-1)
    dl = jax.lax.rsqrt(rowsum)
    dl = jnp.where(jnp.isinf(dl), 0.0, dl)
    es = (1.0 - LAMBDA_COEFF) * dl[:, None] * item_emb
    ab = LAMBDA_COEFF * weight
    h = _item_h(w, dl, image_original_adj, text_original_adj, es, item_emb, ab)

    ego = jnp.concatenate([user_emb, item_emb], axis=0)
    n_nodes = ego.shape[0]
    row, col = edge_index[0], edge_index[1]
    deg = jnp.zeros((n_nodes,), dtype=ego.dtype).at[row].add(1.0)
    deg_inv_sqrt = jnp.power(deg, -0.5)
    deg_inv_sqrt = jnp.where(jnp.isinf(deg_inv_sqrt), 0.0, deg_inv_sqrt)
    norm = deg_inv_sqrt[row] * deg_inv_sqrt[col]
    acc = ego
    cur = ego
    for _ in range(N_LAYERS):
        msg = norm[:, None] * cur[row]
        cur = jax.ops.segment_sum(msg, col, num_segments=n_nodes)
        acc = acc + cur
    all_e = acc / (N_LAYERS + 1)
    u_g = all_e[:NUM_USER]
    i_g = all_e[NUM_USER:]
    h_norm = h / jnp.clip(jnp.linalg.norm(h, axis=1, keepdims=True), 1e-12, None)
    i_g = i_g + h_norm
    return jnp.concatenate([u_g, i_g], axis=0)


# speedup vs baseline: 15.9913x; 12.8128x over previous
import functools

import jax
import jax.numpy as jnp
from jax import lax
from jax.experimental import pallas as pl
from jax.experimental.pallas import tpu as pltpu
from jax.experimental.pallas import tpu_sc as plsc

NUM_USER = 16384
NUM_ITEM = 4096
DIM_E = 64
TOPK = 10
N_LAYERS = 2
LAMBDA_COEFF = 0.9

ROWS_B1 = 256
ROWS_B2 = 256


def _proj_norm_body(v_ref, wv_ref, bv_ref, t_ref, wt_ref, bt_ref, xi_ref, xt_ref):
    fi = jnp.dot(v_ref[:], wv_ref[:], preferred_element_type=jnp.float32) + bv_ref[:]
    ft = jnp.dot(t_ref[:], wt_ref[:], preferred_element_type=jnp.float32) + bt_ref[:]
    xi_ref[:] = fi / jnp.sqrt(jnp.sum(fi * fi, axis=1, keepdims=True))
    xt_ref[:] = ft / jnp.sqrt(jnp.sum(ft * ft, axis=1, keepdims=True))


def _projected_normalized(v_feat, t_feat, Wv, bv, Wt, bt):
    n = v_feat.shape[0]
    rb = 512
    grid = (n // rb,)
    return pl.pallas_call(
        _proj_norm_body,
        grid=grid,
        in_specs=[
            pl.BlockSpec((rb, v_feat.shape[1]), lambda i: (i, 0)),
            pl.BlockSpec((v_feat.shape[1], DIM_E), lambda i: (0, 0)),
            pl.BlockSpec((1, DIM_E), lambda i: (0, 0)),
            pl.BlockSpec((rb, t_feat.shape[1]), lambda i: (i, 0)),
            pl.BlockSpec((t_feat.shape[1], DIM_E), lambda i: (0, 0)),
            pl.BlockSpec((1, DIM_E), lambda i: (0, 0)),
        ],
        out_specs=[
            pl.BlockSpec((rb, DIM_E), lambda i: (i, 0)),
            pl.BlockSpec((rb, DIM_E), lambda i: (i, 0)),
        ],
        out_shape=[
            jax.ShapeDtypeStruct((n, DIM_E), jnp.float32),
            jax.ShapeDtypeStruct((n, DIM_E), jnp.float32),
        ],
    )(v_feat, Wv, bv.reshape(1, DIM_E), t_feat, Wt, bt.reshape(1, DIM_E))


def _row_topk_threshold(sim):
    cur = sim
    for _ in range(TOPK - 1):
        m = jnp.max(cur, axis=1, keepdims=True)
        cur = jnp.where(cur >= m, -jnp.inf, cur)
    return jnp.max(cur, axis=1, keepdims=True)


def _knn_w_body(wvec_ref, xi_blk, xi_all, xt_blk, xt_all, w_out, rowsum_out):
    si = jax.lax.dot_general(xi_blk[:], xi_all[:], (((1,), (1,)), ((), ())),
                             preferred_element_type=jnp.float32)
    st = jax.lax.dot_general(xt_blk[:], xt_all[:], (((1,), (1,)), ((), ())),
                             preferred_element_type=jnp.float32)
    thr_i = _row_topk_threshold(si)
    thr_t = _row_topk_threshold(st)
    w0 = wvec_ref[0, 0]
    w1 = wvec_ref[0, 1]
    w = w0 * jnp.where(si >= thr_i, si, 0.0) + w1 * jnp.where(st >= thr_t, st, 0.0)
    w_out[:] = w
    rowsum_out[0, 0, :] = jnp.sum(w, axis=1)


def _knn_w(xi, xt, wvec):
    n = xi.shape[0]
    grid = (n // ROWS_B1,)
    return pl.pallas_call(
        _knn_w_body,
        grid=grid,
        in_specs=[
            pl.BlockSpec(memory_space=pltpu.SMEM),
            pl.BlockSpec((ROWS_B1, DIM_E), lambda i: (i, 0)),
            pl.BlockSpec((n, DIM_E), lambda i: (0, 0)),
            pl.BlockSpec((ROWS_B1, DIM_E), lambda i: (i, 0)),
            pl.BlockSpec((n, DIM_E), lambda i: (0, 0)),
        ],
        out_specs=[
            pl.BlockSpec((ROWS_B1, n), lambda i: (i, 0)),
            pl.BlockSpec((1, 1, ROWS_B1), lambda i: (i, 0, 0)),
        ],
        out_shape=[
            jax.ShapeDtypeStruct((n, n), jnp.float32),
            jax.ShapeDtypeStruct((n // ROWS_B1, 1, ROWS_B1), jnp.float32),
        ],
    )(wvec.reshape(1, 2), xi, xi, xt, xt)


def _item_h_body(ab_ref, dl_blk, w_blk, io_blk, to_blk, es_ref, e_ref, h_out):
    a = ab_ref[0, 0]
    b = ab_ref[0, 1]
    learned = jnp.dot(w_blk[:], es_ref[:], preferred_element_type=jnp.float32)
    learned = learned * dl_blk[0, 0, :][:, None]
    orig = a * io_blk[:] + b * to_blk[:]
    h_out[:] = learned + jnp.dot(orig, e_ref[:], preferred_element_type=jnp.float32)


def _item_h(w, dl, image_original_adj, text_original_adj, es, item_emb, ab):
    n = w.shape[0]
    grid = (n // ROWS_B2,)
    return pl.pallas_call(
        _item_h_body,
        grid=grid,
        in_specs=[
            pl.BlockSpec(memory_space=pltpu.SMEM),
            pl.BlockSpec((1, 1, ROWS_B2), lambda i: (i, 0, 0)),
            pl.BlockSpec((ROWS_B2, n), lambda i: (i, 0)),
            pl.BlockSpec((ROWS_B2, n), lambda i: (i, 0)),
            pl.BlockSpec((ROWS_B2, n), lambda i: (i, 0)),
            pl.BlockSpec((n, DIM_E), lambda i: (0, 0)),
            pl.BlockSpec((n, DIM_E), lambda i: (0, 0)),
        ],
        out_specs=pl.BlockSpec((ROWS_B2, DIM_E), lambda i: (i, 0)),
        out_shape=jax.ShapeDtypeStruct((n, DIM_E), jnp.float32),
    )(ab.reshape(1, 2), dl.reshape(n // ROWS_B2, 1, ROWS_B2), w,
      image_original_adj, text_original_adj, es, item_emb)


_SC_CORES = 2
_SC_SUBCORES = 16
_EDGE_CHUNK = 128


def _gcn_layer_sc(s_scaled, row, col, zeros_stripe):
    n_nodes, d = s_scaled.shape
    n_edges = row.shape[0]
    n_workers = _SC_CORES * _SC_SUBCORES
    ew = n_edges // n_workers
    nchunk = ew // _EDGE_CHUNK
    stripe = n_nodes // _SC_SUBCORES
    mesh = plsc.VectorSubcoreMesh(core_axis_name="c", subcore_axis_name="s")

    @functools.partial(
        pl.kernel,
        out_type=jax.ShapeDtypeStruct((_SC_CORES, n_nodes, d), jnp.float32),
        mesh=mesh,
        scratch_types=[
            pltpu.VMEM((_EDGE_CHUNK,), jnp.int32),
            pltpu.VMEM((_EDGE_CHUNK,), jnp.int32),
            pltpu.VMEM((_EDGE_CHUNK, d), jnp.float32),
            pltpu.VMEM_SHARED((n_nodes, d), jnp.float32),
            pltpu.SemaphoreType.DMA,
        ],
        compiler_params=pltpu.CompilerParams(use_tc_tiling_on_sc=False),
    )
    def layer(s_hbm, row_hbm, col_hbm, z_hbm, out_hbm, idx_r, idx_c, buf, acc, sem):
        cid = lax.axis_index("c")
        sid = lax.axis_index("s")
        pltpu.sync_copy(z_hbm, acc.at[pl.ds(sid * stripe, stripe)])
        plsc.subcore_barrier()

        base = (cid * _SC_SUBCORES + sid) * ew

        def chunk(k, carry):
            off = base + k * _EDGE_CHUNK
            pltpu.sync_copy(row_hbm.at[pl.ds(off, _EDGE_CHUNK)], idx_r)
            pltpu.sync_copy(col_hbm.at[pl.ds(off, _EDGE_CHUNK)], idx_c)
            pltpu.async_copy(s_hbm.at[idx_r], buf, sem).wait()
            pltpu.sync_copy(buf, acc.at[idx_c], add=True)
            return carry

        lax.fori_loop(0, nchunk, chunk, 0)
        plsc.subcore_barrier()
        r0 = sid * stripe
        pltpu.sync_copy(acc.at[pl.ds(r0, stripe)],
                        out_hbm.at[cid, pl.ds(r0, stripe)])

    return layer(s_scaled, row, col, zeros_stripe)


def kernel(v_feat, t_feat, Wv, bv, Wt, bt, modal_weight, user_emb, item_emb,
           edge_index, image_original_adj, text_original_adj, build_item_graph):
    weight = jax.nn.softmax(modal_weight, axis=0)

    xi, xt = _projected_normalized(v_feat, t_feat, Wv, bv, Wt, bt)
    w, rowsum3 = _knn_w(xi, xt, weight)
    rowsum = rowsum3.reshape(-1)
    dl = jax.lax.rsqrt(rowsum)
    dl = jnp.where(jnp.isinf(dl), 0.0, dl)
    es = (1.0 - LAMBDA_COEFF) * dl[:, None] * item_emb
    ab = LAMBDA_COEFF * weight
    h = _item_h(w, dl, image_original_adj, text_original_adj, es, item_emb, ab)

    ego = jnp.concatenate([user_emb, item_emb], axis=0)
    n_nodes = ego.shape[0]
    row, col = edge_index[0], edge_index[1]
    deg = jnp.zeros((n_nodes,), dtype=ego.dtype).at[row].add(1.0)
    dinv = jax.lax.rsqrt(deg)
    dinv = jnp.where(jnp.isinf(dinv), 0.0, dinv)[:, None]
    zeros_stripe = jnp.zeros((n_nodes // _SC_SUBCORES, DIM_E), jnp.float32)
    acc = ego
    cur = ego
    for _ in range(N_LAYERS):
        p = _gcn_layer_sc(dinv * cur, row, col, zeros_stripe)
        cur = dinv * (p[0] + p[1])
        acc = acc + cur
    all_e = acc / (N_LAYERS + 1)
    u_g = all_e[:NUM_USER]
    i_g = all_e[NUM_USER:]
    h_norm = h / jnp.clip(jnp.linalg.norm(h, axis=1, keepdims=True), 1e-12, None)
    i_g = i_g + h_norm
    return jnp.concatenate([u_g, i_g], axis=0)


# speedup vs baseline: 20.5953x; 1.2879x over previous
import functools

import jax
import jax.numpy as jnp
from jax import lax
from jax.experimental import pallas as pl
from jax.experimental.pallas import tpu as pltpu
from jax.experimental.pallas import tpu_sc as plsc

NUM_USER = 16384
NUM_ITEM = 4096
DIM_E = 64
TOPK = 10
N_LAYERS = 2
LAMBDA_COEFF = 0.9

ROWS_B1 = 256
ROWS_B2 = 256


def _proj_norm_body(v_ref, wv_ref, bv_ref, t_ref, wt_ref, bt_ref, xi_ref, xt_ref):
    fi = jnp.dot(v_ref[:], wv_ref[:], preferred_element_type=jnp.float32) + bv_ref[:]
    ft = jnp.dot(t_ref[:], wt_ref[:], preferred_element_type=jnp.float32) + bt_ref[:]
    xi_ref[:] = fi / jnp.sqrt(jnp.sum(fi * fi, axis=1, keepdims=True))
    xt_ref[:] = ft / jnp.sqrt(jnp.sum(ft * ft, axis=1, keepdims=True))


def _projected_normalized(v_feat, t_feat, Wv, bv, Wt, bt):
    n = v_feat.shape[0]
    rb = 512
    grid = (n // rb,)
    return pl.pallas_call(
        _proj_norm_body,
        grid=grid,
        in_specs=[
            pl.BlockSpec((rb, v_feat.shape[1]), lambda i: (i, 0)),
            pl.BlockSpec((v_feat.shape[1], DIM_E), lambda i: (0, 0)),
            pl.BlockSpec((1, DIM_E), lambda i: (0, 0)),
            pl.BlockSpec((rb, t_feat.shape[1]), lambda i: (i, 0)),
            pl.BlockSpec((t_feat.shape[1], DIM_E), lambda i: (0, 0)),
            pl.BlockSpec((1, DIM_E), lambda i: (0, 0)),
        ],
        out_specs=[
            pl.BlockSpec((rb, DIM_E), lambda i: (i, 0)),
            pl.BlockSpec((rb, DIM_E), lambda i: (i, 0)),
        ],
        out_shape=[
            jax.ShapeDtypeStruct((n, DIM_E), jnp.float32),
            jax.ShapeDtypeStruct((n, DIM_E), jnp.float32),
        ],
    )(v_feat, Wv, bv.reshape(1, DIM_E), t_feat, Wt, bt.reshape(1, DIM_E))


def _row_topk_threshold(sim):
    cur = sim
    for _ in range(TOPK - 1):
        m = jnp.max(cur, axis=1, keepdims=True)
        cur = jnp.where(cur >= m, -jnp.inf, cur)
    return jnp.max(cur, axis=1, keepdims=True)


def _knn_w_body(wvec_ref, xi_blk, xi_all, xt_blk, xt_all, w_out, rowsum_out):
    si = jax.lax.dot_general(xi_blk[:], xi_all[:], (((1,), (1,)), ((), ())),
                             preferred_element_type=jnp.float32)
    st = jax.lax.dot_general(xt_blk[:], xt_all[:], (((1,), (1,)), ((), ())),
                             preferred_element_type=jnp.float32)
    thr_i = _row_topk_threshold(si)
    thr_t = _row_topk_threshold(st)
    w0 = wvec_ref[0, 0]
    w1 = wvec_ref[0, 1]
    w = w0 * jnp.where(si >= thr_i, si, 0.0) + w1 * jnp.where(st >= thr_t, st, 0.0)
    w_out[:] = w
    rowsum_out[0, 0, :] = jnp.sum(w, axis=1)


def _knn_w(xi, xt, wvec):
    n = xi.shape[0]
    grid = (n // ROWS_B1,)
    return pl.pallas_call(
        _knn_w_body,
        grid=grid,
        in_specs=[
            pl.BlockSpec(memory_space=pltpu.SMEM),
            pl.BlockSpec((ROWS_B1, DIM_E), lambda i: (i, 0)),
            pl.BlockSpec((n, DIM_E), lambda i: (0, 0)),
            pl.BlockSpec((ROWS_B1, DIM_E), lambda i: (i, 0)),
            pl.BlockSpec((n, DIM_E), lambda i: (0, 0)),
        ],
        out_specs=[
            pl.BlockSpec((ROWS_B1, n), lambda i: (i, 0)),
            pl.BlockSpec((1, 1, ROWS_B1), lambda i: (i, 0, 0)),
        ],
        out_shape=[
            jax.ShapeDtypeStruct((n, n), jnp.float32),
            jax.ShapeDtypeStruct((n // ROWS_B1, 1, ROWS_B1), jnp.float32),
        ],
    )(wvec.reshape(1, 2), xi, xi, xt, xt)


def _item_h_body(ab_ref, dl_blk, w_blk, io_blk, to_blk, es_ref, e_ref, h_out):
    a = ab_ref[0, 0]
    b = ab_ref[0, 1]
    learned = jnp.dot(w_blk[:], es_ref[:], preferred_element_type=jnp.float32)
    learned = learned * dl_blk[0, 0, :][:, None]
    orig = a * io_blk[:] + b * to_blk[:]
    h_out[:] = learned + jnp.dot(orig, e_ref[:], preferred_element_type=jnp.float32)


def _item_h(w, dl, image_original_adj, text_original_adj, es, item_emb, ab):
    n = w.shape[0]
    grid = (n // ROWS_B2,)
    return pl.pallas_call(
        _item_h_body,
        grid=grid,
        in_specs=[
            pl.BlockSpec(memory_space=pltpu.SMEM),
            pl.BlockSpec((1, 1, ROWS_B2), lambda i: (i, 0, 0)),
            pl.BlockSpec((ROWS_B2, n), lambda i: (i, 0)),
            pl.BlockSpec((ROWS_B2, n), lambda i: (i, 0)),
            pl.BlockSpec((ROWS_B2, n), lambda i: (i, 0)),
            pl.BlockSpec((n, DIM_E), lambda i: (0, 0)),
            pl.BlockSpec((n, DIM_E), lambda i: (0, 0)),
        ],
        out_specs=pl.BlockSpec((ROWS_B2, DIM_E), lambda i: (i, 0)),
        out_shape=jax.ShapeDtypeStruct((n, DIM_E), jnp.float32),
    )(ab.reshape(1, 2), dl.reshape(n // ROWS_B2, 1, ROWS_B2), w,
      image_original_adj, text_original_adj, es, item_emb)


_SC_CORES = 2
_SC_SUBCORES = 16
_EDGE_CHUNK = 128


_NSLOT = 4
_SEG = 32


def _gcn_layer_sc(s_scaled, row2d, col2d, zeros_stripe):
    n_nodes, d = s_scaled.shape
    n_edges = row2d.size
    n_workers = _SC_CORES * _SC_SUBCORES
    ew = n_edges // n_workers
    nchunk = ew // _EDGE_CHUNK
    nseg = nchunk // _SEG
    ngroup = _SEG // _NSLOT
    stripe = n_nodes // _SC_SUBCORES
    mesh = plsc.VectorSubcoreMesh(core_axis_name="c", subcore_axis_name="s")

    @functools.partial(
        pl.kernel,
        out_type=jax.ShapeDtypeStruct((_SC_CORES, n_nodes, d), jnp.float32),
        mesh=mesh,
        scratch_types=[
            pltpu.VMEM((_SEG, _EDGE_CHUNK), jnp.int32),
            pltpu.VMEM((_SEG, _EDGE_CHUNK), jnp.int32),
            [pltpu.VMEM((_EDGE_CHUNK, d), jnp.float32)] * _NSLOT,
            [pltpu.SemaphoreType.DMA] * _NSLOT,
            [pltpu.SemaphoreType.DMA] * _NSLOT,
            pltpu.VMEM_SHARED((n_nodes, d), jnp.float32),
        ],
        compiler_params=pltpu.CompilerParams(use_tc_tiling_on_sc=False),
    )
    def layer(s_hbm, row_hbm, col_hbm, z_hbm, out_hbm,
              idx_r, idx_c, bufs, gsems, ssems, acc):
        cid = lax.axis_index("c")
        sid = lax.axis_index("s")
        pltpu.sync_copy(z_hbm, acc.at[pl.ds(sid * stripe, stripe)])
        plsc.subcore_barrier()

        wid = cid * _SC_SUBCORES + sid
        c0 = wid * nchunk

        def segment(g, carry):
            pltpu.sync_copy(row_hbm.at[pl.ds(c0 + g * _SEG, _SEG)], idx_r)
            pltpu.sync_copy(col_hbm.at[pl.ds(c0 + g * _SEG, _SEG)], idx_c)

            def group(j, carry2):
                k0 = j * _NSLOT
                for s in range(_NSLOT):
                    @pl.when(j > 0)
                    def _():
                        pltpu.make_async_copy(
                            bufs[s], acc.at[idx_c.at[k0 + s]],
                            ssems[s]).wait()
                    pltpu.async_copy(s_hbm.at[idx_r.at[k0 + s]],
                                     bufs[s], gsems[s])
                for s in range(_NSLOT):
                    pltpu.make_async_copy(
                        s_hbm.at[idx_r.at[k0 + s]], bufs[s], gsems[s]).wait()
                    pltpu.async_copy(bufs[s], acc.at[idx_c.at[k0 + s]],
                                     ssems[s], add=True)
                return carry2

            lax.fori_loop(0, ngroup, group, 0)
            for s in range(_NSLOT):
                pltpu.make_async_copy(
                    bufs[s], acc.at[idx_c.at[_SEG - _NSLOT + s]],
                    ssems[s]).wait()
            return carry

        lax.fori_loop(0, nseg, segment, 0)
        plsc.subcore_barrier()
        r0 = sid * stripe
        pltpu.sync_copy(acc.at[pl.ds(r0, stripe)],
                        out_hbm.at[cid, pl.ds(r0, stripe)])

    return layer(s_scaled, row2d, col2d, zeros_stripe)


def kernel(v_feat, t_feat, Wv, bv, Wt, bt, modal_weight, user_emb, item_emb,
           edge_index, image_original_adj, text_original_adj, build_item_graph):
    weight = jax.nn.softmax(modal_weight, axis=0)

    xi, xt = _projected_normalized(v_feat, t_feat, Wv, bv, Wt, bt)
    w, rowsum3 = _knn_w(xi, xt, weight)
    rowsum = rowsum3.reshape(-1)
    dl = jax.lax.rsqrt(rowsum)
    dl = jnp.where(jnp.isinf(dl), 0.0, dl)
    es = (1.0 - LAMBDA_COEFF) * dl[:, None] * item_emb
    ab = LAMBDA_COEFF * weight
    h = _item_h(w, dl, image_original_adj, text_original_adj, es, item_emb, ab)

    ego = jnp.concatenate([user_emb, item_emb], axis=0)
    n_nodes = ego.shape[0]
    row, col = edge_index[0], edge_index[1]
    deg = jnp.zeros((n_nodes,), dtype=ego.dtype).at[row].add(1.0)
    dinv = jax.lax.rsqrt(deg)
    dinv = jnp.where(jnp.isinf(dinv), 0.0, dinv)[:, None]
    row2d = row.reshape(-1, _EDGE_CHUNK)
    col2d = col.reshape(-1, _EDGE_CHUNK)
    zeros_stripe = jnp.zeros((n_nodes // _SC_SUBCORES, DIM_E), jnp.float32)
    acc = ego
    cur = ego
    for _ in range(N_LAYERS):
        p = _gcn_layer_sc(dinv * cur, row2d, col2d, zeros_stripe)
        cur = dinv * (p[0] + p[1])
        acc = acc + cur
    all_e = acc / (N_LAYERS + 1)
    u_g = all_e[:NUM_USER]
    i_g = all_e[NUM_USER:]
    h_norm = h / jnp.clip(jnp.linalg.norm(h, axis=1, keepdims=True), 1e-12, None)
    i_g = i_g + h_norm
    return jnp.concatenate([u_g, i_g], axis=0)


# speedup vs baseline: 36.8875x; 1.7911x over previous
import functools

import jax
import jax.numpy as jnp
from jax import lax
from jax.experimental import pallas as pl
from jax.experimental.pallas import tpu as pltpu
from jax.experimental.pallas import tpu_sc as plsc

NUM_USER = 16384
NUM_ITEM = 4096
DIM_E = 64
TOPK = 10
N_LAYERS = 2
LAMBDA_COEFF = 0.9

ROWS_B1 = 256
ROWS_B2 = 256


def _proj_norm_body(v_ref, wv_ref, bv_ref, t_ref, wt_ref, bt_ref, xi_ref, xt_ref):
    fi = jnp.dot(v_ref[:], wv_ref[:], preferred_element_type=jnp.float32) + bv_ref[:]
    ft = jnp.dot(t_ref[:], wt_ref[:], preferred_element_type=jnp.float32) + bt_ref[:]
    xi_ref[:] = fi / jnp.sqrt(jnp.sum(fi * fi, axis=1, keepdims=True))
    xt_ref[:] = ft / jnp.sqrt(jnp.sum(ft * ft, axis=1, keepdims=True))


def _projected_normalized(v_feat, t_feat, Wv, bv, Wt, bt):
    n = v_feat.shape[0]
    rb = 512
    grid = (n // rb,)
    return pl.pallas_call(
        _proj_norm_body,
        grid=grid,
        in_specs=[
            pl.BlockSpec((rb, v_feat.shape[1]), lambda i: (i, 0)),
            pl.BlockSpec((v_feat.shape[1], DIM_E), lambda i: (0, 0)),
            pl.BlockSpec((1, DIM_E), lambda i: (0, 0)),
            pl.BlockSpec((rb, t_feat.shape[1]), lambda i: (i, 0)),
            pl.BlockSpec((t_feat.shape[1], DIM_E), lambda i: (0, 0)),
            pl.BlockSpec((1, DIM_E), lambda i: (0, 0)),
        ],
        out_specs=[
            pl.BlockSpec((rb, DIM_E), lambda i: (i, 0)),
            pl.BlockSpec((rb, DIM_E), lambda i: (i, 0)),
        ],
        out_shape=[
            jax.ShapeDtypeStruct((n, DIM_E), jnp.float32),
            jax.ShapeDtypeStruct((n, DIM_E), jnp.float32),
        ],
    )(v_feat, Wv, bv.reshape(1, DIM_E), t_feat, Wt, bt.reshape(1, DIM_E))


def _row_topk_threshold(sim):
    cur = sim
    for _ in range(TOPK - 1):
        m = jnp.max(cur, axis=1, keepdims=True)
        cur = jnp.where(cur >= m, -jnp.inf, cur)
    return jnp.max(cur, axis=1, keepdims=True)


def _knn_w_body(wvec_ref, xi_blk, xi_all, xt_blk, xt_all, w_out, rowsum_out):
    si = jax.lax.dot_general(xi_blk[:], xi_all[:], (((1,), (1,)), ((), ())),
                             preferred_element_type=jnp.float32)
    st = jax.lax.dot_general(xt_blk[:], xt_all[:], (((1,), (1,)), ((), ())),
                             preferred_element_type=jnp.float32)
    thr_i = _row_topk_threshold(si)
    thr_t = _row_topk_threshold(st)
    w0 = wvec_ref[0, 0]
    w1 = wvec_ref[0, 1]
    w = w0 * jnp.where(si >= thr_i, si, 0.0) + w1 * jnp.where(st >= thr_t, st, 0.0)
    w_out[:] = w
    rowsum_out[0, 0, :] = jnp.sum(w, axis=1)


def _knn_w(xi, xt, wvec):
    n = xi.shape[0]
    grid = (n // ROWS_B1,)
    return pl.pallas_call(
        _knn_w_body,
        grid=grid,
        in_specs=[
            pl.BlockSpec(memory_space=pltpu.SMEM),
            pl.BlockSpec((ROWS_B1, DIM_E), lambda i: (i, 0)),
            pl.BlockSpec((n, DIM_E), lambda i: (0, 0)),
            pl.BlockSpec((ROWS_B1, DIM_E), lambda i: (i, 0)),
            pl.BlockSpec((n, DIM_E), lambda i: (0, 0)),
        ],
        out_specs=[
            pl.BlockSpec((ROWS_B1, n), lambda i: (i, 0)),
            pl.BlockSpec((1, 1, ROWS_B1), lambda i: (i, 0, 0)),
        ],
        out_shape=[
            jax.ShapeDtypeStruct((n, n), jnp.float32),
            jax.ShapeDtypeStruct((n // ROWS_B1, 1, ROWS_B1), jnp.float32),
        ],
    )(wvec.reshape(1, 2), xi, xi, xt, xt)


def _item_h_body(ab_ref, dl_blk, w_blk, io_blk, to_blk, es_ref, e_ref, h_out):
    a = ab_ref[0, 0]
    b = ab_ref[0, 1]
    learned = jnp.dot(w_blk[:], es_ref[:], preferred_element_type=jnp.float32)
    learned = learned * dl_blk[0, 0, :][:, None]
    orig = a * io_blk[:] + b * to_blk[:]
    h_out[:] = learned + jnp.dot(orig, e_ref[:], preferred_element_type=jnp.float32)


def _item_h(w, dl, image_original_adj, text_original_adj, es, item_emb, ab):
    n = w.shape[0]
    grid = (n // ROWS_B2,)
    return pl.pallas_call(
        _item_h_body,
        grid=grid,
        in_specs=[
            pl.BlockSpec(memory_space=pltpu.SMEM),
            pl.BlockSpec((1, 1, ROWS_B2), lambda i: (i, 0, 0)),
            pl.BlockSpec((ROWS_B2, n), lambda i: (i, 0)),
            pl.BlockSpec((ROWS_B2, n), lambda i: (i, 0)),
            pl.BlockSpec((ROWS_B2, n), lambda i: (i, 0)),
            pl.BlockSpec((n, DIM_E), lambda i: (0, 0)),
            pl.BlockSpec((n, DIM_E), lambda i: (0, 0)),
        ],
        out_specs=pl.BlockSpec((ROWS_B2, DIM_E), lambda i: (i, 0)),
        out_shape=jax.ShapeDtypeStruct((n, DIM_E), jnp.float32),
    )(ab.reshape(1, 2), dl.reshape(n // ROWS_B2, 1, ROWS_B2), w,
      image_original_adj, text_original_adj, es, item_emb)


_SC_CORES = 2
_SC_SUBCORES = 16
_EDGE_CHUNK = 128


_NSLOT = 4
_SEG = 32


def _gcn_layer_sc(s_scaled, row2d, col2d, zeros_stripe):
    n_nodes, d = s_scaled.shape
    n_edges = row2d.size
    n_workers = _SC_CORES * _SC_SUBCORES
    ew = n_edges // n_workers
    nchunk = ew // _EDGE_CHUNK
    nseg = nchunk // _SEG
    ngroup = _SEG // _NSLOT
    stripe = n_nodes // _SC_SUBCORES
    mesh = plsc.VectorSubcoreMesh(core_axis_name="c", subcore_axis_name="s")

    @functools.partial(
        pl.kernel,
        out_type=jax.ShapeDtypeStruct((_SC_CORES, n_nodes, d), jnp.float32),
        mesh=mesh,
        scratch_types=[
            pltpu.VMEM((_SEG, _EDGE_CHUNK), jnp.int32),
            pltpu.VMEM((_SEG, _EDGE_CHUNK), jnp.int32),
            [pltpu.VMEM((_EDGE_CHUNK, d), jnp.float32)] * _NSLOT,
            [pltpu.SemaphoreType.DMA] * _NSLOT,
            [pltpu.SemaphoreType.DMA] * _NSLOT,
            pltpu.VMEM_SHARED((n_nodes, d), jnp.float32),
        ],
        compiler_params=pltpu.CompilerParams(use_tc_tiling_on_sc=False),
    )
    def layer(s_hbm, row_hbm, col_hbm, z_hbm, out_hbm,
              idx_r, idx_c, bufs, gsems, ssems, acc):
        cid = lax.axis_index("c")
        sid = lax.axis_index("s")
        pltpu.sync_copy(z_hbm, acc.at[pl.ds(sid * stripe, stripe)])
        plsc.subcore_barrier()

        wid = cid * _SC_SUBCORES + sid
        c0 = wid * nchunk

        def segment(g, carry):
            pltpu.sync_copy(row_hbm.at[pl.ds(c0 + g * _SEG, _SEG)], idx_r)
            pltpu.sync_copy(col_hbm.at[pl.ds(c0 + g * _SEG, _SEG)], idx_c)

            def group(j, carry2):
                k0 = j * _NSLOT
                for s in range(_NSLOT):
                    @pl.when(j > 0)
                    def _():
                        pltpu.make_async_copy(
                            bufs[s], acc.at[idx_c.at[k0 + s]],
                            ssems[s]).wait()
                    pltpu.async_copy(s_hbm.at[idx_r.at[k0 + s]],
                                     bufs[s], gsems[s])
                for s in range(_NSLOT):
                    pltpu.make_async_copy(
                        s_hbm.at[idx_r.at[k0 + s]], bufs[s], gsems[s]).wait()
                    pltpu.async_copy(bufs[s], acc.at[idx_c.at[k0 + s]],
                                     ssems[s], add=True)
                return carry2

            lax.fori_loop(0, ngroup, group, 0)
            for s in range(_NSLOT):
                pltpu.make_async_copy(
                    bufs[s], acc.at[idx_c.at[_SEG - _NSLOT + s]],
                    ssems[s]).wait()
            return carry

        lax.fori_loop(0, nseg, segment, 0)
        plsc.subcore_barrier()
        r0 = sid * stripe
        pltpu.sync_copy(acc.at[pl.ds(r0, stripe)],
                        out_hbm.at[cid, pl.ds(r0, stripe)])

    return layer(s_scaled, row2d, col2d, zeros_stripe)


_DEG_W = 16


def _deg_sc(row2d, n_nodes):
    n_edges = row2d.size
    n_workers = _SC_CORES * _SC_SUBCORES
    nchunk = n_edges // n_workers // _EDGE_CHUNK
    stripe = n_nodes // _SC_SUBCORES
    mesh = plsc.VectorSubcoreMesh(core_axis_name="c", subcore_axis_name="s")

    @functools.partial(
        pl.kernel,
        out_type=jax.ShapeDtypeStruct((_SC_CORES, n_nodes, _DEG_W), jnp.float32),
        mesh=mesh,
        scratch_types=[
            pltpu.VMEM((nchunk, _EDGE_CHUNK), jnp.int32),
            pltpu.VMEM((_EDGE_CHUNK, _DEG_W), jnp.float32),
            pltpu.VMEM((_EDGE_CHUNK, _DEG_W), jnp.float32),
            pltpu.VMEM_SHARED((n_nodes, _DEG_W), jnp.float32),
            pltpu.SemaphoreType.DMA,
        ],
        compiler_params=pltpu.CompilerParams(use_tc_tiling_on_sc=False),
    )
    def deg_kernel(row_hbm, out_hbm, idx_r, ones_buf, zer_buf, acc_view, sem):
        cid = lax.axis_index("c")
        sid = lax.axis_index("s")
        ones16 = jnp.ones((_DEG_W,), jnp.float32)
        zero16 = jnp.zeros((_DEG_W,), jnp.float32)

        def fill(i, carry):
            ones_buf[i, :] = ones16
            zer_buf[i, :] = zero16
            return carry

        lax.fori_loop(0, _EDGE_CHUNK, fill, 0)
        for t in range(stripe // _EDGE_CHUNK):
            pltpu.sync_copy(zer_buf,
                            acc_view.at[pl.ds(sid * stripe + t * _EDGE_CHUNK,
                                              _EDGE_CHUNK)])
        plsc.subcore_barrier()
        wid = cid * _SC_SUBCORES + sid
        pltpu.sync_copy(row_hbm.at[pl.ds(wid * nchunk, nchunk)], idx_r)

        def chunk(k, carry):
            pltpu.async_copy(ones_buf, acc_view.at[idx_r.at[k]], sem, add=True)
            return carry

        lax.fori_loop(0, nchunk, chunk, 0)

        def drain(k, carry):
            pltpu.make_async_copy(ones_buf, acc_view.at[idx_r.at[0]],
                                  sem).wait()
            return carry

        lax.fori_loop(0, nchunk, drain, 0)
        plsc.subcore_barrier()
        r0 = sid * stripe
        pltpu.sync_copy(acc_view.at[pl.ds(r0, stripe)],
                        out_hbm.at[cid, pl.ds(r0, stripe)])

    return deg_kernel(row2d)


def kernel(v_feat, t_feat, Wv, bv, Wt, bt, modal_weight, user_emb, item_emb,
           edge_index, image_original_adj, text_original_adj, build_item_graph):
    weight = jax.nn.softmax(modal_weight, axis=0)

    xi, xt = _projected_normalized(v_feat, t_feat, Wv, bv, Wt, bt)
    w, rowsum3 = _knn_w(xi, xt, weight)
    rowsum = rowsum3.reshape(-1)
    dl = jax.lax.rsqrt(rowsum)
    dl = jnp.where(jnp.isinf(dl), 0.0, dl)
    es = (1.0 - LAMBDA_COEFF) * dl[:, None] * item_emb
    ab = LAMBDA_COEFF * weight
    h = _item_h(w, dl, image_original_adj, text_original_adj, es, item_emb, ab)

    ego = jnp.concatenate([user_emb, item_emb], axis=0)
    n_nodes = ego.shape[0]
    row, col = edge_index[0], edge_index[1]
    row2d = row.reshape(-1, _EDGE_CHUNK)
    col2d = col.reshape(-1, _EDGE_CHUNK)
    degp = _deg_sc(row2d, n_nodes)
    deg = degp[0, :, 0] + degp[1, :, 0]
    dinv = jax.lax.rsqrt(deg)
    dinv = jnp.where(jnp.isinf(dinv), 0.0, dinv)[:, None]
    zeros_stripe = jnp.zeros((n_nodes // _SC_SUBCORES, DIM_E), jnp.float32)
    acc = ego
    cur = ego
    for _ in range(N_LAYERS):
        p = _gcn_layer_sc(dinv * cur, row2d, col2d, zeros_stripe)
        cur = dinv * (p[0] + p[1])
        acc = acc + cur
    all_e = acc / (N_LAYERS + 1)
    u_g = all_e[:NUM_USER]
    i_g = all_e[NUM_USER:]
    h_norm = h / jnp.clip(jnp.linalg.norm(h, axis=1, keepdims=True), 1e-12, None)
    i_g = i_g + h_norm
    return jnp.concatenate([u_g, i_g], axis=0)


# speedup vs baseline: 37.6939x; 1.0219x over previous
import functools

import jax
import jax.numpy as jnp
from jax import lax
from jax.experimental import pallas as pl
from jax.experimental.pallas import tpu as pltpu
from jax.experimental.pallas import tpu_sc as plsc

NUM_USER = 16384
NUM_ITEM = 4096
DIM_E = 64
TOPK = 10
N_LAYERS = 2
LAMBDA_COEFF = 0.9

ROWS_B1 = 256
ROWS_B2 = 256


def _proj_norm_body(v_ref, wv_ref, bv_ref, t_ref, wt_ref, bt_ref, xi_ref, xt_ref):
    fi = jnp.dot(v_ref[:], wv_ref[:], preferred_element_type=jnp.float32) + bv_ref[:]
    ft = jnp.dot(t_ref[:], wt_ref[:], preferred_element_type=jnp.float32) + bt_ref[:]
    xi_ref[:] = fi / jnp.sqrt(jnp.sum(fi * fi, axis=1, keepdims=True))
    xt_ref[:] = ft / jnp.sqrt(jnp.sum(ft * ft, axis=1, keepdims=True))


def _projected_normalized(v_feat, t_feat, Wv, bv, Wt, bt):
    n = v_feat.shape[0]
    rb = 512
    grid = (n // rb,)
    return pl.pallas_call(
        _proj_norm_body,
        grid=grid,
        in_specs=[
            pl.BlockSpec((rb, v_feat.shape[1]), lambda i: (i, 0)),
            pl.BlockSpec((v_feat.shape[1], DIM_E), lambda i: (0, 0)),
            pl.BlockSpec((1, DIM_E), lambda i: (0, 0)),
            pl.BlockSpec((rb, t_feat.shape[1]), lambda i: (i, 0)),
            pl.BlockSpec((t_feat.shape[1], DIM_E), lambda i: (0, 0)),
            pl.BlockSpec((1, DIM_E), lambda i: (0, 0)),
        ],
        out_specs=[
            pl.BlockSpec((rb, DIM_E), lambda i: (i, 0)),
            pl.BlockSpec((rb, DIM_E), lambda i: (i, 0)),
        ],
        out_shape=[
            jax.ShapeDtypeStruct((n, DIM_E), jnp.float32),
            jax.ShapeDtypeStruct((n, DIM_E), jnp.float32),
        ],
    )(v_feat, Wv, bv.reshape(1, DIM_E), t_feat, Wt, bt.reshape(1, DIM_E))


def _row_topk_threshold(sim):
    cur = sim
    for _ in range(TOPK - 1):
        m = jnp.max(cur, axis=1, keepdims=True)
        cur = jnp.where(cur >= m, -jnp.inf, cur)
    return jnp.max(cur, axis=1, keepdims=True)


def _knn_w_body(wvec_ref, xi_blk, xi_all, xt_blk, xt_all, w_out, rowsum_out):
    si = jax.lax.dot_general(xi_blk[:], xi_all[:], (((1,), (1,)), ((), ())),
                             preferred_element_type=jnp.float32)
    st = jax.lax.dot_general(xt_blk[:], xt_all[:], (((1,), (1,)), ((), ())),
                             preferred_element_type=jnp.float32)
    thr_i = _row_topk_threshold(si)
    thr_t = _row_topk_threshold(st)
    w0 = wvec_ref[0, 0]
    w1 = wvec_ref[0, 1]
    w = w0 * jnp.where(si >= thr_i, si, 0.0) + w1 * jnp.where(st >= thr_t, st, 0.0)
    w_out[:] = w.astype(jnp.bfloat16)
    rowsum_out[0, 0, :] = jnp.sum(w, axis=1)


def _knn_w(xi, xt, wvec):
    n = xi.shape[0]
    grid = (n // ROWS_B1,)
    return pl.pallas_call(
        _knn_w_body,
        grid=grid,
        in_specs=[
            pl.BlockSpec(memory_space=pltpu.SMEM),
            pl.BlockSpec((ROWS_B1, DIM_E), lambda i: (i, 0)),
            pl.BlockSpec((n, DIM_E), lambda i: (0, 0)),
            pl.BlockSpec((ROWS_B1, DIM_E), lambda i: (i, 0)),
            pl.BlockSpec((n, DIM_E), lambda i: (0, 0)),
        ],
        out_specs=[
            pl.BlockSpec((ROWS_B1, n), lambda i: (i, 0)),
            pl.BlockSpec((1, 1, ROWS_B1), lambda i: (i, 0, 0)),
        ],
        out_shape=[
            jax.ShapeDtypeStruct((n, n), jnp.bfloat16),
            jax.ShapeDtypeStruct((n // ROWS_B1, 1, ROWS_B1), jnp.float32),
        ],
    )(wvec.reshape(1, 2), xi, xi, xt, xt)


def _item_h_body(ab_ref, dl_blk, w_blk, io_blk, to_blk, es_ref, e_ref, h_out):
    a = ab_ref[0, 0]
    b = ab_ref[0, 1]
    learned = jnp.dot(w_blk[:], es_ref[:], preferred_element_type=jnp.float32)
    learned = learned * dl_blk[0, 0, :][:, None]
    orig = a * io_blk[:] + b * to_blk[:]
    h_out[:] = learned + jnp.dot(orig, e_ref[:], preferred_element_type=jnp.float32)


def _item_h(w, dl, image_original_adj, text_original_adj, es, item_emb, ab):
    n = w.shape[0]
    grid = (n // ROWS_B2,)
    return pl.pallas_call(
        _item_h_body,
        grid=grid,
        in_specs=[
            pl.BlockSpec(memory_space=pltpu.SMEM),
            pl.BlockSpec((1, 1, ROWS_B2), lambda i: (i, 0, 0)),
            pl.BlockSpec((ROWS_B2, n), lambda i: (i, 0)),
            pl.BlockSpec((ROWS_B2, n), lambda i: (i, 0)),
            pl.BlockSpec((ROWS_B2, n), lambda i: (i, 0)),
            pl.BlockSpec((n, DIM_E), lambda i: (0, 0)),
            pl.BlockSpec((n, DIM_E), lambda i: (0, 0)),
        ],
        out_specs=pl.BlockSpec((ROWS_B2, DIM_E), lambda i: (i, 0)),
        out_shape=jax.ShapeDtypeStruct((n, DIM_E), jnp.float32),
    )(ab.reshape(1, 2), dl.reshape(n // ROWS_B2, 1, ROWS_B2), w,
      image_original_adj, text_original_adj, es.astype(jnp.bfloat16), item_emb)


_SC_CORES = 2
_SC_SUBCORES = 16
_EDGE_CHUNK = 128


_NSLOT = 4
_SEG = 32


def _gcn_layer_sc(s_scaled, row2d, col2d, zeros_stripe):
    n_nodes, d = s_scaled.shape
    n_edges = row2d.size
    n_workers = _SC_CORES * _SC_SUBCORES
    ew = n_edges // n_workers
    nchunk = ew // _EDGE_CHUNK
    nseg = nchunk // _SEG
    ngroup = _SEG // _NSLOT
    stripe = n_nodes // _SC_SUBCORES
    mesh = plsc.VectorSubcoreMesh(core_axis_name="c", subcore_axis_name="s")

    @functools.partial(
        pl.kernel,
        out_type=jax.ShapeDtypeStruct((_SC_CORES, n_nodes, d), jnp.float32),
        mesh=mesh,
        scratch_types=[
            pltpu.VMEM((_SEG, _EDGE_CHUNK), jnp.int32),
            pltpu.VMEM((_SEG, _EDGE_CHUNK), jnp.int32),
            [pltpu.VMEM((_EDGE_CHUNK, d), jnp.float32)] * _NSLOT,
            [pltpu.SemaphoreType.DMA] * _NSLOT,
            [pltpu.SemaphoreType.DMA] * _NSLOT,
            pltpu.VMEM_SHARED((n_nodes, d), jnp.float32),
        ],
        compiler_params=pltpu.CompilerParams(use_tc_tiling_on_sc=False),
    )
    def layer(s_hbm, row_hbm, col_hbm, z_hbm, out_hbm,
              idx_r, idx_c, bufs, gsems, ssems, acc):
        cid = lax.axis_index("c")
        sid = lax.axis_index("s")
        pltpu.sync_copy(z_hbm, acc.at[pl.ds(sid * stripe, stripe)])
        plsc.subcore_barrier()

        wid = cid * _SC_SUBCORES + sid
        c0 = wid * nchunk

        def segment(g, carry):
            pltpu.sync_copy(row_hbm.at[pl.ds(c0 + g * _SEG, _SEG)], idx_r)
            pltpu.sync_copy(col_hbm.at[pl.ds(c0 + g * _SEG, _SEG)], idx_c)

            def group(j, carry2):
                k0 = j * _NSLOT
                for s in range(_NSLOT):
                    @pl.when(j > 0)
                    def _():
                        pltpu.make_async_copy(
                            bufs[s], acc.at[idx_c.at[k0 + s]],
                            ssems[s]).wait()
                    pltpu.async_copy(s_hbm.at[idx_r.at[k0 + s]],
                                     bufs[s], gsems[s])
                for s in range(_NSLOT):
                    pltpu.make_async_copy(
                        s_hbm.at[idx_r.at[k0 + s]], bufs[s], gsems[s]).wait()
                    pltpu.async_copy(bufs[s], acc.at[idx_c.at[k0 + s]],
                                     ssems[s], add=True)
                return carry2

            lax.fori_loop(0, ngroup, group, 0)
            for s in range(_NSLOT):
                pltpu.make_async_copy(
                    bufs[s], acc.at[idx_c.at[_SEG - _NSLOT + s]],
                    ssems[s]).wait()
            return carry

        lax.fori_loop(0, nseg, segment, 0)
        plsc.subcore_barrier()
        r0 = sid * stripe
        pltpu.sync_copy(acc.at[pl.ds(r0, stripe)],
                        out_hbm.at[cid, pl.ds(r0, stripe)])

    return layer(s_scaled, row2d, col2d, zeros_stripe)


_DEG_W = 16


def _deg_sc(row2d, n_nodes):
    n_edges = row2d.size
    n_workers = _SC_CORES * _SC_SUBCORES
    nchunk = n_edges // n_workers // _EDGE_CHUNK
    stripe = n_nodes // _SC_SUBCORES
    mesh = plsc.VectorSubcoreMesh(core_axis_name="c", subcore_axis_name="s")

    @functools.partial(
        pl.kernel,
        out_type=jax.ShapeDtypeStruct((_SC_CORES, n_nodes, _DEG_W), jnp.float32),
        mesh=mesh,
        scratch_types=[
            pltpu.VMEM((nchunk, _EDGE_CHUNK), jnp.int32),
            pltpu.VMEM((_EDGE_CHUNK, _DEG_W), jnp.float32),
            pltpu.VMEM((_EDGE_CHUNK, _DEG_W), jnp.float32),
            pltpu.VMEM_SHARED((n_nodes, _DEG_W), jnp.float32),
            pltpu.SemaphoreType.DMA,
        ],
        compiler_params=pltpu.CompilerParams(use_tc_tiling_on_sc=False),
    )
    def deg_kernel(row_hbm, out_hbm, idx_r, ones_buf, zer_buf, acc_view, sem):
        cid = lax.axis_index("c")
        sid = lax.axis_index("s")
        ones16 = jnp.ones((_DEG_W,), jnp.float32)
        zero16 = jnp.zeros((_DEG_W,), jnp.float32)

        def fill(i, carry):
            ones_buf[i, :] = ones16
            zer_buf[i, :] = zero16
            return carry

        lax.fori_loop(0, _EDGE_CHUNK, fill, 0)
        for t in range(stripe // _EDGE_CHUNK):
            pltpu.sync_copy(zer_buf,
                            acc_view.at[pl.ds(sid * stripe + t * _EDGE_CHUNK,
                                              _EDGE_CHUNK)])
        plsc.subcore_barrier()
        wid = cid * _SC_SUBCORES + sid
        pltpu.sync_copy(row_hbm.at[pl.ds(wid * nchunk, nchunk)], idx_r)

        def chunk(k, carry):
            pltpu.async_copy(ones_buf, acc_view.at[idx_r.at[k]], sem, add=True)
            return carry

        lax.fori_loop(0, nchunk, chunk, 0)

        def drain(k, carry):
            pltpu.make_async_copy(ones_buf, acc_view.at[idx_r.at[0]],
                                  sem).wait()
            return carry

        lax.fori_loop(0, nchunk, drain, 0)
        plsc.subcore_barrier()
        r0 = sid * stripe
        pltpu.sync_copy(acc_view.at[pl.ds(r0, stripe)],
                        out_hbm.at[cid, pl.ds(r0, stripe)])

    return deg_kernel(row2d)


def kernel(v_feat, t_feat, Wv, bv, Wt, bt, modal_weight, user_emb, item_emb,
           edge_index, image_original_adj, text_original_adj, build_item_graph):
    weight = jax.nn.softmax(modal_weight, axis=0)

    ego = jnp.concatenate([user_emb, item_emb], axis=0)
    n_nodes = ego.shape[0]
    row, col = edge_index[0], edge_index[1]
    row2d = row.reshape(-1, _EDGE_CHUNK)
    col2d = col.reshape(-1, _EDGE_CHUNK)
    degp = _deg_sc(row2d, n_nodes)
    deg = degp[0, :, 0] + degp[1, :, 0]
    dinv = jax.lax.rsqrt(deg)
    dinv = jnp.where(jnp.isinf(dinv), 0.0, dinv)[:, None]
    zeros_stripe = jnp.zeros((n_nodes // _SC_SUBCORES, DIM_E), jnp.float32)
    acc = ego
    cur = ego
    for _ in range(N_LAYERS):
        p = _gcn_layer_sc(dinv * cur, row2d, col2d, zeros_stripe)
        cur = dinv * (p[0] + p[1])
        acc = acc + cur
    all_e = acc / (N_LAYERS + 1)

    xi, xt = _projected_normalized(v_feat, t_feat, Wv, bv, Wt, bt)
    w, rowsum3 = _knn_w(xi, xt, weight)
    rowsum = rowsum3.reshape(-1)
    dl = jax.lax.rsqrt(rowsum)
    dl = jnp.where(jnp.isinf(dl), 0.0, dl)
    es = (1.0 - LAMBDA_COEFF) * dl[:, None] * item_emb
    ab = LAMBDA_COEFF * weight
    h = _item_h(w, dl, image_original_adj, text_original_adj, es, item_emb, ab)
    u_g = all_e[:NUM_USER]
    i_g = all_e[NUM_USER:]
    h_norm = h / jnp.clip(jnp.linalg.norm(h, axis=1, keepdims=True), 1e-12, None)
    i_g = i_g + h_norm
    return jnp.concatenate([u_g, i_g], axis=0)
